# baseline (speedup 1.0000x reference)
import functools

import jax
import jax.numpy as jnp
from jax import lax
from jax.experimental import pallas as pl
from jax.experimental.pallas import tpu as pltpu
from jax.experimental.pallas import tpu_sc as plsc

B, S, D = 4096, 50, 128
NC, NS = 2, 16
NW = NC * NS
NSPLIT = 2
BH = B // NSPLIT
SEQ_W = BH // NW
NCH = SEQ_W
NB = 8
SLACK = 3


def _make_emb_kernel():
    mesh = plsc.VectorSubcoreMesh(core_axis_name="c", subcore_axis_name="s")

    @functools.partial(
        pl.kernel,
        mesh=mesh,
        out_type=jax.ShapeDtypeStruct((BH, S, D), jnp.float32),
        scratch_types=(
            [pltpu.VMEM((SEQ_W, S), jnp.int32)]
            + [pltpu.VMEM((S, D), jnp.float32) for _ in range(NB)]
            + [pltpu.SemaphoreType.DMA for _ in range(2 * NB)]
        ),
    )
    def emb(idx_hbm, table_hbm, out_hbm, idx_v, *rest):
        bufs = rest[:NB]
        gsem = rest[NB:2 * NB]
        ssem = rest[2 * NB:]
        wid = lax.axis_index("s") * NC + lax.axis_index("c")
        pltpu.sync_copy(idx_hbm.at[wid], idx_v)
        seq_base = wid * SEQ_W

        def gather_copy(j, b):
            return pltpu.make_async_copy(
                table_hbm.at[idx_v.at[j]], bufs[b], gsem[b])

        def store_copy(j, b):
            return pltpu.make_async_copy(
                bufs[b], out_hbm.at[seq_base + j], ssem[b])

        def step(j, i, mid):
            gather_copy(j, i).wait()
            store_copy(j, i).start()
            if mid:
                bp = (i - SLACK) % NB
                store_copy(j - SLACK, bp).wait()
                gather_copy(j + NB - SLACK, bp).start()

        for jj in range(NB):
            gather_copy(jj, jj).start()
        for i in range(NB):
            step(i, i, i >= SLACK)

        def group(g, c):
            j0 = g * NB
            for i in range(NB):
                step(j0 + i, i, True)
            return c

        lax.fori_loop(1, NCH // NB - 1, group, 0)

        j0 = NCH - NB
        for i in range(NB):
            step(j0 + i, i, i < SLACK)
        for i in range(NB):
            store_copy(j0 + i, i).wait()

    return emb


_emb = _make_emb_kernel()


def kernel(input_ids, table):
    idx = input_ids.astype(jnp.int32)
    outs = [
        _emb(idx[k * BH:(k + 1) * BH].reshape(NW, SEQ_W, S), table)
        for k in range(NSPLIT)
    ]
    return jnp.concatenate(outs, axis=0)


# speedup vs baseline: 2.8882x; 2.8882x over previous
import functools

import jax
import jax.numpy as jnp
from jax import lax
from jax.experimental import pallas as pl
from jax.experimental.pallas import tpu as pltpu
from jax.experimental.pallas import tpu_sc as plsc

B, S, D = 4096, 50, 128
NC, NS = 2, 16
NW = NC * NS
BW = B // NW
CW = 64
CPP = BW // CW
NCH = S * CPP
NB = 10
SLACK = 3


def _make_emb_kernel():
    mesh = plsc.VectorSubcoreMesh(core_axis_name="c", subcore_axis_name="s")

    @functools.partial(
        pl.kernel,
        mesh=mesh,
        out_type=jax.ShapeDtypeStruct((S, B, D), jnp.float32),
        scratch_types=(
            [pltpu.VMEM((S, BW), jnp.int32)]
            + [pltpu.VMEM((CW, D), jnp.float32) for _ in range(NB)]
            + [pltpu.SemaphoreType.DMA for _ in range(2 * NB)]
        ),
    )
    def emb(idx_hbm, table_hbm, out_hbm, idx_v, *rest):
        bufs = rest[:NB]
        gsem = rest[NB:2 * NB]
        ssem = rest[2 * NB:]
        wid = lax.axis_index("s") * NC + lax.axis_index("c")
        b0 = wid * BW
        pltpu.sync_copy(idx_hbm.at[:, pl.ds(b0, BW)], idx_v)

        def gather_copy(j, b):
            return pltpu.make_async_copy(
                table_hbm.at[idx_v.at[j // CPP, pl.ds((j % CPP) * CW, CW)]],
                bufs[b], gsem[b])

        def store_copy(j, b):
            return pltpu.make_async_copy(
                bufs[b], out_hbm.at[j // CPP, pl.ds(b0 + (j % CPP) * CW, CW)],
                ssem[b])

        def step(j, i, mid):
            gather_copy(j, i).wait()
            store_copy(j, i).start()
            if mid:
                bp = (i - SLACK) % NB
                store_copy(j - SLACK, bp).wait()
                gather_copy(j + NB - SLACK, bp).start()

        for jj in range(NB):
            gather_copy(jj, jj).start()
        for i in range(NB):
            step(i, i, i >= SLACK)

        def group(g, c):
            j0 = g * NB
            for i in range(NB):
                step(j0 + i, i, True)
            return c

        lax.fori_loop(1, NCH // NB - 1, group, 0)

        j0 = NCH - NB
        for i in range(NB):
            step(j0 + i, i, i < SLACK)
        for i in range(NB):
            store_copy(j0 + i, i).wait()

    return emb


_emb = _make_emb_kernel()


def kernel(input_ids, table):
    idx_t = input_ids.astype(jnp.int32).T
    out_t = _emb(idx_t, table)
    return out_t.transpose(1, 0, 2)
